# baseline (speedup 1.0000x reference)
import functools
import jax
import jax.numpy as jnp
from jax import lax
from jax.experimental import pallas as pl
from jax.experimental.pallas import tpu as pltpu
from jax.experimental.pallas import tpu_sc as plsc

_N = 50000
_F = 256
_H = 4
_S = 512
_SA = 32
_SB = 16
_R = 2048
_NB = 25
_NPAD = _R * _NB
_NEG = -1e30


def _shift_right(v, d, fill):
    pad = jnp.full(v.shape[:-1] + (d,), fill, v.dtype)
    return jnp.concatenate([pad, v[..., :-d]], axis=-1)


def _seg_cummax(v, ids):
    cur = ids
    d = 1
    while d < _R:
        same = _shift_right(cur, d, jnp.int32(-1)) == cur
        vs = _shift_right(v, d, jnp.float32(_NEG))
        v = jnp.where(same, jnp.maximum(v, vs), v)
        d *= 2
    return v


def _dot(a, b, dims):
    return jax.lax.dot_general(a, b, (dims, ((), ())),
                               preferred_element_type=jnp.float32)


def _gather_T(stat, oa, ob):
    rows = []
    for h in range(_H):
        u = _dot(stat[h * _SA:(h + 1) * _SA], oa, ((0,), (0,)))
        rows.append(jnp.sum(u * ob, axis=0, keepdims=True))
    return jnp.concatenate(rows, axis=0)


def _scatter_T(vals, oa, ob):
    lhs = jnp.concatenate([oa * vals[h:h + 1] for h in range(_H)], axis=0)
    return _dot(lhs, ob, ((1,), (1,)))


def _masks(ids):
    oa = (jax.lax.broadcasted_iota(jnp.int32, (_SA, _R), 0)
          == (ids >> 4)).astype(jnp.float32)
    ob = (jax.lax.broadcasted_iota(jnp.int32, (_SB, _R), 0)
          == (ids & 15)).astype(jnp.float32)
    return oa, ob


def _stats_body(x_ref, ids_ref, w1_ref, b1_ref, w2_ref, b2c_ref,
                e_out, f_out, m_sc, d_sc, mp_sc):
    i = pl.program_id(0)
    ids = ids_ref[...].reshape(1, _R)
    valid = ids < _S
    oa, ob = _masks(ids)
    w_eff = jnp.dot(w1_ref[...], w2_ref[...],
                    preferred_element_type=jnp.float32)
    beff = _dot(w2_ref[...], b1_ref[...], ((0,), (1,)))
    s_t = _dot(w_eff, x_ref[...], ((0,), (1,))) + beff + b2c_ref[...]
    v = jnp.where(valid, s_t, _NEG)
    cmax = _seg_cummax(v, ids)
    nxt = jnp.concatenate(
        [ids[:, 1:], jnp.full((1, 1), -1, jnp.int32)], axis=1)
    last = (ids != nxt).astype(jnp.float32)
    m_blk = _scatter_T(cmax * last, oa, ob)
    pres = _dot(oa * last, ob, ((1,), (1,)))
    pres = jnp.concatenate([pres] * _H, axis=0)
    m_blk = jnp.where(pres > 0, m_blk, _NEG)
    mg = _gather_T(m_blk, oa, ob)
    e = jnp.where(valid, jnp.exp(s_t - mg), 0.0)
    e_out[...] = e.reshape(1, _H, _R)
    mp_sc[i] = m_blk
    sum_blk = _scatter_T(e, oa, ob)

    @pl.when(i == 0)
    def _():
        m_sc[...] = jnp.full((_H * _SA, _SB), _NEG, jnp.float32)
        d_sc[...] = jnp.zeros((_H * _SA, _SB), jnp.float32)

    pm = m_sc[...]
    pd = d_sc[...]
    mn = jnp.maximum(pm, m_blk)
    d_sc[...] = pd * jnp.exp(pm - mn) + sum_blk * jnp.exp(m_blk - mn)
    m_sc[...] = mn

    @pl.when(i == _NB - 1)
    def _():
        mg = m_sc[...][None]
        dg = d_sc[...][None]
        f_out[...] = jnp.exp(mp_sc[...] - mg) / (dg + 1e-16)


def _sc_body(idx_hbm, f_hbm, g0, g1, g2, g3,
             i0, i1, i2, i3, f0, f1, f2, f3, sem):
    cid = lax.axis_index("c")
    sid = lax.axis_index("s")
    wid = sid * 2 + cid
    i_vs = [i0, i1, i2, i3]
    f_vs = [f0, f1, f2, f3]
    g_hs = [g0, g1, g2, g3]
    for b in range(_NB):
        @pl.when(wid == b % 32)
        def _(b=b):
            for h in range(_H):
                pltpu.sync_copy(
                    idx_hbm.at[pl.ds((h * _NB + b) * _R, _R)], i_vs[h])
            copies = [pltpu.async_copy(f_hbm.at[i_vs[h]], f_vs[h], sem)
                      for h in range(_H)]
            for c in copies:
                c.wait()
            for h in range(_H):
                pltpu.sync_copy(f_vs[h], g_hs[h].at[pl.ds(b * _R, _R)])


def _scale_body(x_ref, e_ref, f_ref, out_ref):
    w = jnp.sum(e_ref[0] * f_ref[0], axis=0, keepdims=True)
    out_ref[...] = x_ref[...] * w.reshape(_R, 1)


@jax.jit
def kernel(x, batch, W1, b1, W2, b2):
    ids_pad = jnp.pad(batch.astype(jnp.int32), (0, _NPAD - _N),
                      constant_values=_S)
    ids3 = ids_pad.reshape(_NB, 1, _R)
    idc = jnp.minimum(ids_pad, _S - 1).reshape(_NB, _R)
    idx = (jnp.arange(_H, dtype=jnp.int32)[:, None, None] * _S
           + jnp.arange(_NB, dtype=jnp.int32)[None, :, None] * (_H * _S)
           + idc[None]).reshape(-1)
    b1r = b1.reshape(1, _F)
    b2c = b2.reshape(_H, 1)

    e3, f3 = pl.pallas_call(
        _stats_body,
        grid=(_NB,),
        in_specs=[
            pl.BlockSpec((_R, _F), lambda i: (i, 0)),
            pl.BlockSpec((1, 1, _R), lambda i: (i, 0, 0)),
            pl.BlockSpec((_F, _F), lambda i: (0, 0)),
            pl.BlockSpec((1, _F), lambda i: (0, 0)),
            pl.BlockSpec((_F, _H), lambda i: (0, 0)),
            pl.BlockSpec((_H, 1), lambda i: (0, 0)),
        ],
        out_specs=[
            pl.BlockSpec((1, _H, _R), lambda i: (i, 0, 0)),
            pl.BlockSpec((_NB, _H * _SA, _SB), lambda i: (0, 0, 0)),
        ],
        out_shape=[
            jax.ShapeDtypeStruct((_NB, _H, _R), jnp.float32),
            jax.ShapeDtypeStruct((_NB, _H * _SA, _SB), jnp.float32),
        ],
        scratch_shapes=[
            pltpu.VMEM((_H * _SA, _SB), jnp.float32),
            pltpu.VMEM((_H * _SA, _SB), jnp.float32),
            pltpu.VMEM((_NB, _H * _SA, _SB), jnp.float32),
        ],
    )(x, ids3, W1, b1r, W2, b2c)

    sc_fn = functools.partial(
        pl.kernel,
        out_type=[jax.ShapeDtypeStruct((_NPAD,), jnp.float32)
                  for _ in range(_H)],
        mesh=plsc.VectorSubcoreMesh(core_axis_name="c", subcore_axis_name="s"),
        scratch_types=(
            [pltpu.VMEM((_R,), jnp.int32) for _ in range(_H)]
            + [pltpu.VMEM((_R,), jnp.float32) for _ in range(_H)]
            + [pltpu.SemaphoreType.DMA]
        ),
    )(_sc_body)
    fgs = sc_fn(idx, f3.reshape(-1))
    f4 = jnp.stack([f.reshape(_NB, _R) for f in fgs], axis=1)

    out = pl.pallas_call(
        _scale_body,
        grid=(_NB,),
        in_specs=[
            pl.BlockSpec((_R, _F), lambda i: (i, 0)),
            pl.BlockSpec((1, _H, _R), lambda i: (i, 0, 0)),
            pl.BlockSpec((1, _H, _R), lambda i: (i, 0, 0)),
        ],
        out_specs=pl.BlockSpec((_R, _F), lambda i: (i, 0)),
        out_shape=jax.ShapeDtypeStruct((_N, _F), jnp.float32),
    )(x, e3, f4)
    return out


# speedup vs baseline: 1.0081x; 1.0081x over previous
import functools
import jax
import jax.numpy as jnp
from jax import lax
from jax.experimental import pallas as pl
from jax.experimental.pallas import tpu as pltpu
from jax.experimental.pallas import tpu_sc as plsc

_N = 50000
_F = 256
_H = 4
_S = 512
_SA = 32
_SB = 16
_R = 2048
_NB = 25
_NPAD = _R * _NB
_NEG = -1e30


def _shift_right(v, d, fill):
    pad = jnp.full(v.shape[:-1] + (d,), fill, v.dtype)
    return jnp.concatenate([pad, v[..., :-d]], axis=-1)


def _seg_cummax(v, ids):
    cur = ids
    d = 1
    while d < _R:
        same = _shift_right(cur, d, jnp.int32(-1)) == cur
        vs = _shift_right(v, d, jnp.float32(_NEG))
        v = jnp.where(same, jnp.maximum(v, vs), v)
        d *= 2
    return v


def _dot(a, b, dims):
    return jax.lax.dot_general(a, b, (dims, ((), ())),
                               preferred_element_type=jnp.float32)


def _gather_T(stat, oa, ob):
    rows = []
    for h in range(_H):
        u = _dot(stat[h * _SA:(h + 1) * _SA], oa, ((0,), (0,)))
        rows.append(jnp.sum(u * ob, axis=0, keepdims=True))
    return jnp.concatenate(rows, axis=0)


def _scatter_T(vals, oa, ob):
    lhs = jnp.concatenate([oa * vals[h:h + 1] for h in range(_H)], axis=0)
    return _dot(lhs, ob, ((1,), (1,)))


def _masks(ids):
    oa = (jax.lax.broadcasted_iota(jnp.int32, (_SA, _R), 0)
          == (ids >> 4)).astype(jnp.float32)
    ob = (jax.lax.broadcasted_iota(jnp.int32, (_SB, _R), 0)
          == (ids & 15)).astype(jnp.float32)
    return oa, ob


def _stats_body(x_ref, ids_ref, w1_ref, b1_ref, w2_ref, b2c_ref,
                e_out, f_out, m_sc, d_sc, mp_sc):
    i = pl.program_id(0)
    ids = ids_ref[...].reshape(1, _R)
    valid = ids < _S
    oa, ob = _masks(ids)
    w_eff = jnp.dot(w1_ref[...], w2_ref[...],
                    preferred_element_type=jnp.float32)
    beff = _dot(w2_ref[...], b1_ref[...], ((0,), (1,)))
    s_t = _dot(w_eff, x_ref[...], ((0,), (1,))) + beff + b2c_ref[...]
    v = jnp.where(valid, s_t, _NEG)
    cmax = _seg_cummax(v, ids)
    nxt = jnp.concatenate(
        [ids[:, 1:], jnp.full((1, 1), -1, jnp.int32)], axis=1)
    last = (ids != nxt).astype(jnp.float32)
    m_blk = _scatter_T(cmax * last, oa, ob)
    pres = _dot(oa * last, ob, ((1,), (1,)))
    pres = jnp.concatenate([pres] * _H, axis=0)
    m_blk = jnp.where(pres > 0, m_blk, _NEG)
    mg = _gather_T(m_blk, oa, ob)
    e = jnp.where(valid, jnp.exp(s_t - mg), 0.0)
    e_out[...] = e.reshape(1, _H, _R)
    mp_sc[i] = m_blk
    sum_blk = _scatter_T(e, oa, ob)

    @pl.when(i == 0)
    def _():
        m_sc[...] = jnp.full((_H * _SA, _SB), _NEG, jnp.float32)
        d_sc[...] = jnp.zeros((_H * _SA, _SB), jnp.float32)

    pm = m_sc[...]
    pd = d_sc[...]
    mn = jnp.maximum(pm, m_blk)
    d_sc[...] = pd * jnp.exp(pm - mn) + sum_blk * jnp.exp(m_blk - mn)
    m_sc[...] = mn

    @pl.when(i == _NB - 1)
    def _():
        mg = m_sc[...][None]
        dg = d_sc[...][None]
        f_out[...] = jnp.exp(mp_sc[...] - mg) / (dg + 1e-16)


def _sc_body(idx_hbm, f_hbm, g0, g1, g2, g3,
             i0, i1, i2, i3, f0, f1, f2, f3, sem):
    cid = lax.axis_index("c")
    sid = lax.axis_index("s")
    wid = sid * 2 + cid
    i_vs = [i0, i1, i2, i3]
    f_vs = [f0, f1, f2, f3]
    g_hs = [g0, g1, g2, g3]
    for t in range(_NB * _H):
        b, h = t // _H, t % _H
        k = (t // 32) % _H

        @pl.when(wid == t % 32)
        def _(b=b, h=h, k=k):
            pltpu.sync_copy(
                idx_hbm.at[pl.ds((h * _NB + b) * _R, _R)], i_vs[k])
            pltpu.async_copy(f_hbm.at[i_vs[k]], f_vs[k], sem).wait()
            pltpu.sync_copy(f_vs[k], g_hs[h].at[pl.ds(b * _R, _R)])


def _scale_body(x_ref, e_ref, f_ref, out_ref):
    w = jnp.sum(e_ref[0] * f_ref[0], axis=0, keepdims=True)
    out_ref[...] = x_ref[...] * w.reshape(_R, 1)


@jax.jit
def kernel(x, batch, W1, b1, W2, b2):
    ids_pad = jnp.pad(batch.astype(jnp.int32), (0, _NPAD - _N),
                      constant_values=_S)
    ids3 = ids_pad.reshape(_NB, 1, _R)
    idc = jnp.minimum(ids_pad, _S - 1).reshape(_NB, _R)
    idx = (jnp.arange(_H, dtype=jnp.int32)[:, None, None] * _S
           + jnp.arange(_NB, dtype=jnp.int32)[None, :, None] * (_H * _S)
           + idc[None]).reshape(-1)
    b1r = b1.reshape(1, _F)
    b2c = b2.reshape(_H, 1)

    e3, f3 = pl.pallas_call(
        _stats_body,
        grid=(_NB,),
        in_specs=[
            pl.BlockSpec((_R, _F), lambda i: (i, 0)),
            pl.BlockSpec((1, 1, _R), lambda i: (i, 0, 0)),
            pl.BlockSpec((_F, _F), lambda i: (0, 0)),
            pl.BlockSpec((1, _F), lambda i: (0, 0)),
            pl.BlockSpec((_F, _H), lambda i: (0, 0)),
            pl.BlockSpec((_H, 1), lambda i: (0, 0)),
        ],
        out_specs=[
            pl.BlockSpec((1, _H, _R), lambda i: (i, 0, 0)),
            pl.BlockSpec((_NB, _H * _SA, _SB), lambda i: (0, 0, 0)),
        ],
        out_shape=[
            jax.ShapeDtypeStruct((_NB, _H, _R), jnp.float32),
            jax.ShapeDtypeStruct((_NB, _H * _SA, _SB), jnp.float32),
        ],
        scratch_shapes=[
            pltpu.VMEM((_H * _SA, _SB), jnp.float32),
            pltpu.VMEM((_H * _SA, _SB), jnp.float32),
            pltpu.VMEM((_NB, _H * _SA, _SB), jnp.float32),
        ],
    )(x, ids3, W1, b1r, W2, b2c)

    sc_fn = functools.partial(
        pl.kernel,
        out_type=[jax.ShapeDtypeStruct((_NPAD,), jnp.float32)
                  for _ in range(_H)],
        mesh=plsc.VectorSubcoreMesh(core_axis_name="c", subcore_axis_name="s"),
        scratch_types=(
            [pltpu.VMEM((_R,), jnp.int32) for _ in range(_H)]
            + [pltpu.VMEM((_R,), jnp.float32) for _ in range(_H)]
            + [pltpu.SemaphoreType.DMA]
        ),
    )(_sc_body)
    fgs = sc_fn(idx, f3.reshape(-1))
    f4 = jnp.stack([f.reshape(_NB, _R) for f in fgs], axis=1)

    out = pl.pallas_call(
        _scale_body,
        grid=(_NB,),
        in_specs=[
            pl.BlockSpec((_R, _F), lambda i: (i, 0)),
            pl.BlockSpec((1, _H, _R), lambda i: (i, 0, 0)),
            pl.BlockSpec((1, _H, _R), lambda i: (i, 0, 0)),
        ],
        out_specs=pl.BlockSpec((_R, _F), lambda i: (i, 0)),
        out_shape=jax.ShapeDtypeStruct((_N, _F), jnp.float32),
    )(x, e3, f4)
    return out
